# Initial kernel scaffold; baseline (speedup 1.0000x reference)
import jax
import jax.numpy as jnp
from jax.experimental import pallas as pl


def kernel(x, edge_index, batch, lysine_mask, W1, b1, W2, b2, W3, b3, W4, b4, w_att, b_att, W_out, b_out):
    raise NotImplementedError("write your pallas kernel here")


# speedup vs baseline: 15.1200x; 15.1200x over previous
import functools

import jax
import jax.numpy as jnp
from jax import lax
from jax.experimental import pallas as pl
from jax.experimental.pallas import tpu as pltpu
from jax.experimental.pallas import tpu_sc as plsc

N = 10000
E = 320000
D = 128
B = 16
NP = 10240
NSC = 2
NSUB = 16
NW = NSC * NSUB
EPT = E // NW
K = 80
NCHUNK = EPT // K
RPT = NP // NSUB
f32 = jnp.float32

_HI = lax.Precision.HIGHEST

_mesh = plsc.VectorSubcoreMesh(core_axis_name="c", subcore_axis_name="s",
                               num_cores=NSC, num_subcores=NSUB)


def _deg_body(dst_hbm, zeros_hbm, ones_hbm, out_hbm, accum, idx_v, ones_v):
    c = lax.axis_index("c")
    s = lax.axis_index("s")
    wid = c * NSUB + s
    pltpu.sync_copy(zeros_hbm, accum.at[pl.ds(s * RPT, RPT)])
    pltpu.sync_copy(dst_hbm.at[wid], idx_v)
    pltpu.sync_copy(ones_hbm, ones_v)
    plsc.subcore_barrier()

    def chunk(j, carry):
        pltpu.sync_copy(ones_v, accum.at[idx_v.at[j]], add=True)
        return carry

    lax.fori_loop(0, NCHUNK, chunk, 0)
    plsc.subcore_barrier()
    pltpu.sync_copy(accum.at[pl.ds(s * RPT, RPT)],
                    out_hbm.at[c, pl.ds(s * RPT, RPT)])


_deg_call = pl.kernel(
    _deg_body,
    out_type=jax.ShapeDtypeStruct((NSC, NP), f32),
    mesh=_mesh,
    scratch_types=[
        pltpu.VMEM_SHARED((NP,), f32),
        pltpu.VMEM((NCHUNK, K), jnp.int32),
        pltpu.VMEM((K,), f32),
    ],
)


def _mp_body(g_hbm, src_hbm, dst_hbm, zeros_hbm, out_hbm,
             accum, src_v, dst_v, rowbuf):
    c = lax.axis_index("c")
    s = lax.axis_index("s")
    wid = c * NSUB + s
    pltpu.sync_copy(zeros_hbm, accum.at[pl.ds(s * RPT, RPT)])
    pltpu.sync_copy(src_hbm.at[wid], src_v)
    pltpu.sync_copy(dst_hbm.at[wid], dst_v)
    plsc.subcore_barrier()

    def chunk(j, carry):
        pltpu.sync_copy(g_hbm.at[src_v.at[j]], rowbuf)
        pltpu.sync_copy(rowbuf, accum.at[dst_v.at[j]], add=True)
        return carry

    lax.fori_loop(0, NCHUNK, chunk, 0)
    plsc.subcore_barrier()
    pltpu.sync_copy(accum.at[pl.ds(s * RPT, RPT)],
                    out_hbm.at[c, pl.ds(s * RPT, RPT)])


_mp_call = pl.kernel(
    _mp_body,
    out_type=jax.ShapeDtypeStruct((NSC, NP, D), f32),
    mesh=_mesh,
    scratch_types=[
        pltpu.VMEM_SHARED((NP, D), f32),
        pltpu.VMEM((NCHUNK, K), jnp.int32),
        pltpu.VMEM((NCHUNK, K), jnp.int32),
        pltpu.VMEM((K, D), f32),
    ],
)


def _pre_body(x_ref, deg_ref, w_ref, dinv_ref, g_ref):
    degp = deg_ref[...]
    dinv = lax.rsqrt(degp[0] + degp[1] + 1.0)
    dinv_ref[...] = dinv
    hw = jnp.dot(x_ref[...], w_ref[...],
                 preferred_element_type=f32, precision=_HI)
    g_ref[...] = hw * dinv


_pre_call = pl.pallas_call(
    _pre_body,
    out_shape=[jax.ShapeDtypeStruct((NP, 1), f32),
               jax.ShapeDtypeStruct((NP, D), f32)],
)


def _mid_body(s_ref, g_ref, dinv_ref, b_ref, w_ref, out_ref):
    sacc = s_ref[...]
    dinv = dinv_ref[...]
    h = jnp.maximum(dinv * (sacc[0] + sacc[1] + g_ref[...]) + b_ref[...], 0.0)
    out_ref[...] = jnp.dot(h, w_ref[...],
                           preferred_element_type=f32, precision=_HI) * dinv


_mid_call = pl.pallas_call(
    _mid_body,
    out_shape=jax.ShapeDtypeStruct((NP, D), f32),
)


def _fin_body(s_ref, g_ref, dinv_ref, b_ref, batch_ref, mask_ref,
              watt_ref, batt_ref, wout_ref, bout_ref, out_ref):
    sacc = s_ref[...]
    h = jnp.maximum(dinv_ref[...] * (sacc[0] + sacc[1] + g_ref[...])
                    + b_ref[...], 0.0)
    bt = batch_ref[...]
    iot = lax.broadcasted_iota(jnp.int32, (B, NP), 0)
    mb = bt == iot
    mf = mb.astype(f32)
    counts = jnp.sum(mf, axis=1, keepdims=True)
    pooled = jnp.dot(mf, h, preferred_element_type=f32,
                     precision=_HI) / jnp.maximum(counts, 1.0)
    protein = pooled / jnp.sqrt(counts + 1e-6)
    scores = lax.dot_general(watt_ref[...], h, (((0,), (1,)), ((), ())),
                             preferred_element_type=f32,
                             precision=_HI) + batt_ref[...]
    mk = mask_ref[...] > 0
    neg = jnp.float32(-jnp.inf)
    masked = jnp.where(mb & mk, scores, neg)
    seg_max = jnp.max(masked, axis=1, keepdims=True)
    seg_max = jnp.where(seg_max > neg, seg_max, 0.0)
    smax_node = jnp.sum(mf * seg_max, axis=0, keepdims=True)
    ex = jnp.where(mk, jnp.exp(scores - smax_node), 0.0)
    seg_sum = jnp.sum(mf * ex, axis=1, keepdims=True)
    denom = jnp.where(seg_sum > 0, seg_sum, 1.0)
    den_node = jnp.sum(mf * denom, axis=0, keepdims=True)
    attn = ex / jnp.maximum(den_node, 1e-37)
    lys = jnp.dot(mf * attn, h, preferred_element_type=f32, precision=_HI)
    out_ref[...] = jnp.dot(protein + lys, wout_ref[...],
                           preferred_element_type=f32,
                           precision=_HI) + bout_ref[...]


_fin_call = pl.pallas_call(
    _fin_body,
    out_shape=jax.ShapeDtypeStruct((B, D), f32),
)


def kernel(x, edge_index, batch, lysine_mask, W1, b1, W2, b2, W3, b3,
           W4, b4, w_att, b_att, W_out, b_out):
    src3 = edge_index[0].reshape(NW, NCHUNK, K)
    dst3 = edge_index[1].reshape(NW, NCHUNK, K)
    xp = jnp.pad(x, ((0, NP - N), (0, 0)))
    batch_t = jnp.pad(batch, (0, NP - N), constant_values=B).reshape(1, NP)
    mask_t = jnp.pad(lysine_mask, (0, NP - N)).astype(f32).reshape(1, NP)
    zeros1 = jnp.zeros((RPT,), f32)
    ones1 = jnp.ones((K,), f32)
    zeros_r = jnp.zeros((RPT, D), f32)

    deg2 = _deg_call(dst3, zeros1, ones1).reshape(NSC, NP, 1)
    dinv, g = _pre_call(xp, deg2, W1)
    for b_l, w_next in ((b1, W2), (b2, W3), (b3, W4)):
        sacc = _mp_call(g, src3, dst3, zeros_r)
        g = _mid_call(sacc, g, dinv, b_l.reshape(1, D), w_next)
    sacc = _mp_call(g, src3, dst3, zeros_r)
    return _fin_call(sacc, g, dinv, b4.reshape(1, D), batch_t, mask_t,
                     w_att, b_att.reshape(1, 1), W_out, b_out.reshape(1, D))


# speedup vs baseline: 21.9636x; 1.4526x over previous
import functools

import jax
import jax.numpy as jnp
from jax import lax
from jax.experimental import pallas as pl
from jax.experimental.pallas import tpu as pltpu
from jax.experimental.pallas import tpu_sc as plsc

N = 10000
E = 320000
D = 128
B = 16
NP = 10240
NSC = 2
NSUB = 16
NW = NSC * NSUB
EPT = E // NW
K = 128
NCHUNK = 80
EPT2 = NCHUNK * K
NBUF = 2
WIN = 16
NWIN = NCHUNK // WIN
RPT = NP // NSUB
f32 = jnp.float32

_HI = lax.Precision.HIGHEST

_mesh = plsc.VectorSubcoreMesh(core_axis_name="c", subcore_axis_name="s",
                               num_cores=NSC, num_subcores=NSUB)


def _deg_body(dst_hbm, zeros_hbm, ones_hbm, out_hbm, accum, idx_v, ones_v):
    c = lax.axis_index("c")
    s = lax.axis_index("s")
    wid = c * NSUB + s
    pltpu.sync_copy(zeros_hbm, accum.at[pl.ds(s * RPT, RPT)])
    pltpu.sync_copy(dst_hbm.at[wid], idx_v)
    pltpu.sync_copy(ones_hbm, ones_v)
    plsc.subcore_barrier()

    def chunk(j, carry):
        pltpu.sync_copy(ones_v, accum.at[idx_v.at[j]], add=True)
        return carry

    lax.fori_loop(0, NCHUNK, chunk, 0)
    plsc.subcore_barrier()
    pltpu.sync_copy(accum.at[pl.ds(s * RPT, RPT)],
                    out_hbm.at[c, pl.ds(s * RPT, RPT)])


_deg_call = pl.kernel(
    _deg_body,
    out_type=jax.ShapeDtypeStruct((NSC, NP), f32),
    mesh=_mesh,
    scratch_types=[
        pltpu.VMEM_SHARED((NP,), f32),
        pltpu.VMEM((NCHUNK, K), jnp.int32),
        pltpu.VMEM((K,), f32),
    ],
)


def _mp_body(g_hbm, src_hbm, dst_hbm, zeros_hbm, out_hbm,
             accum, srcwin, dstwin, rowbuf, gsem, ssem):
    c = lax.axis_index("c")
    s = lax.axis_index("s")
    wid = c * NSUB + s
    pltpu.sync_copy(zeros_hbm, accum.at[pl.ds(s * RPT, RPT)])
    pltpu.sync_copy(src_hbm.at[wid, pl.ds(0, WIN)], srcwin.at[0])
    pltpu.sync_copy(dst_hbm.at[wid, pl.ds(0, WIN)], dstwin.at[0])
    plsc.subcore_barrier()

    pltpu.async_copy(g_hbm.at[srcwin.at[0, 0]], rowbuf.at[0], gsem)

    def window(w, carry):
        wslot = lax.rem(w, 2)
        nslot = lax.rem(w + 1, 2)

        @pl.when(w > 0)
        def _():
            pltpu.make_async_copy(
                rowbuf.at[lax.rem(w * WIN - 1, NBUF)],
                accum.at[dstwin.at[nslot, WIN - 1]], ssem).wait()

        @pl.when(w + 1 < NWIN)
        def _():
            pltpu.sync_copy(src_hbm.at[wid, pl.ds((w + 1) * WIN, WIN)],
                            srcwin.at[nslot])
            pltpu.sync_copy(dst_hbm.at[wid, pl.ds((w + 1) * WIN, WIN)],
                            dstwin.at[nslot])

        for b in range(WIN):
            j = w * WIN + b
            slot = lax.rem(j, NBUF)
            pltpu.make_async_copy(g_hbm.at[srcwin.at[wslot, b]],
                                  rowbuf.at[slot], gsem).wait()
            pltpu.async_copy(rowbuf.at[slot], accum.at[dstwin.at[wslot, b]],
                             ssem, add=True)
            if b > 0:
                pltpu.make_async_copy(rowbuf.at[lax.rem(j - 1, NBUF)],
                                      accum.at[dstwin.at[wslot, b - 1]],
                                      ssem).wait()

            @pl.when(j + 1 < NCHUNK)
            def _():
                gw = lax.rem((j + 1) // WIN, 2)
                pltpu.async_copy(
                    g_hbm.at[srcwin.at[gw, (b + 1) % WIN]],
                    rowbuf.at[lax.rem(j + 1, NBUF)], gsem)

        return carry

    lax.fori_loop(0, NWIN, window, 0)
    pltpu.make_async_copy(rowbuf.at[(NCHUNK - 1) % NBUF],
                          accum.at[dstwin.at[(NWIN - 1) % 2, WIN - 1]],
                          ssem).wait()
    plsc.subcore_barrier()
    pltpu.sync_copy(accum.at[pl.ds(s * RPT, RPT)],
                    out_hbm.at[c, pl.ds(s * RPT, RPT)])


_mp_call = pl.kernel(
    _mp_body,
    out_type=jax.ShapeDtypeStruct((NSC, NP, D), f32),
    mesh=_mesh,
    scratch_types=[
        pltpu.VMEM_SHARED((NP, D), f32),
        pltpu.VMEM((2, WIN, K), jnp.int32),
        pltpu.VMEM((2, WIN, K), jnp.int32),
        pltpu.VMEM((NBUF, K, D), f32),
        pltpu.SemaphoreType.DMA,
        pltpu.SemaphoreType.DMA,
    ],
)


def _pre_body(x_ref, deg_ref, w_ref, dinv_ref, g_ref):
    degp = deg_ref[...]
    dinv = lax.rsqrt(degp[0] + degp[1] + 1.0)
    dinv_ref[...] = dinv
    hw = jnp.dot(x_ref[...], w_ref[...],
                 preferred_element_type=f32, precision=_HI)
    g_ref[...] = hw * dinv


_pre_call = pl.pallas_call(
    _pre_body,
    out_shape=[jax.ShapeDtypeStruct((NP, 1), f32),
               jax.ShapeDtypeStruct((NP, D), f32)],
)


def _mid_body(s_ref, g_ref, dinv_ref, b_ref, w_ref, out_ref):
    sacc = s_ref[...]
    dinv = dinv_ref[...]
    h = jnp.maximum(dinv * (sacc[0] + sacc[1] + g_ref[...]) + b_ref[...], 0.0)
    out_ref[...] = jnp.dot(h, w_ref[...],
                           preferred_element_type=f32, precision=_HI) * dinv


_mid_call = pl.pallas_call(
    _mid_body,
    out_shape=jax.ShapeDtypeStruct((NP, D), f32),
)


def _fin_body(s_ref, g_ref, dinv_ref, b_ref, batch_ref, mask_ref,
              watt_ref, batt_ref, wout_ref, bout_ref, out_ref):
    sacc = s_ref[...]
    h = jnp.maximum(dinv_ref[...] * (sacc[0] + sacc[1] + g_ref[...])
                    + b_ref[...], 0.0)
    bt = batch_ref[...]
    iot = lax.broadcasted_iota(jnp.int32, (B, NP), 0)
    mb = bt == iot
    mf = mb.astype(f32)
    counts = jnp.sum(mf, axis=1, keepdims=True)
    pooled = jnp.dot(mf, h, preferred_element_type=f32,
                     precision=_HI) / jnp.maximum(counts, 1.0)
    protein = pooled / jnp.sqrt(counts + 1e-6)
    scores = lax.dot_general(watt_ref[...], h, (((0,), (1,)), ((), ())),
                             preferred_element_type=f32,
                             precision=_HI) + batt_ref[...]
    mk = mask_ref[...] > 0
    neg = jnp.float32(-jnp.inf)
    masked = jnp.where(mb & mk, scores, neg)
    seg_max = jnp.max(masked, axis=1, keepdims=True)
    seg_max = jnp.where(seg_max > neg, seg_max, 0.0)
    smax_node = jnp.sum(mf * seg_max, axis=0, keepdims=True)
    ex = jnp.where(mk, jnp.exp(scores - smax_node), 0.0)
    seg_sum = jnp.sum(mf * ex, axis=1, keepdims=True)
    denom = jnp.where(seg_sum > 0, seg_sum, 1.0)
    den_node = jnp.sum(mf * denom, axis=0, keepdims=True)
    attn = ex / jnp.maximum(den_node, 1e-37)
    lys = jnp.dot(mf * attn, h, preferred_element_type=f32, precision=_HI)
    out_ref[...] = jnp.dot(protein + lys, wout_ref[...],
                           preferred_element_type=f32,
                           precision=_HI) + bout_ref[...]


_fin_call = pl.pallas_call(
    _fin_body,
    out_shape=jax.ShapeDtypeStruct((B, D), f32),
)


def kernel(x, edge_index, batch, lysine_mask, W1, b1, W2, b2, W3, b3,
           W4, b4, w_att, b_att, W_out, b_out):
    npad_rows = NP - N
    pad_iota = jnp.arange(EPT2 - EPT, dtype=jnp.int32)
    pad_src = jnp.broadcast_to(N + (pad_iota % npad_rows), (NW, EPT2 - EPT))
    pad_dst = jnp.broadcast_to(N + ((pad_iota + npad_rows // 2) % npad_rows),
                               (NW, EPT2 - EPT))
    src3 = jnp.concatenate(
        [edge_index[0].reshape(NW, EPT), pad_src], axis=1).reshape(
        NW, NCHUNK, K)
    dst3 = jnp.concatenate(
        [edge_index[1].reshape(NW, EPT), pad_dst], axis=1).reshape(
        NW, NCHUNK, K)
    xp = jnp.pad(x, ((0, NP - N), (0, 0)))
    batch_t = jnp.pad(batch, (0, NP - N), constant_values=B).reshape(1, NP)
    mask_t = jnp.pad(lysine_mask, (0, NP - N)).astype(f32).reshape(1, NP)
    zeros1 = jnp.zeros((RPT,), f32)
    ones1 = jnp.ones((K,), f32)
    zeros_r = jnp.zeros((RPT, D), f32)

    deg2 = _deg_call(dst3, zeros1, ones1).reshape(NSC, NP, 1)
    dinv, g = _pre_call(xp, deg2, W1)
    for b_l, w_next in ((b1, W2), (b2, W3), (b3, W4)):
        sacc = _mp_call(g, src3, dst3, zeros_r)
        g = _mid_call(sacc, g, dinv, b_l.reshape(1, D), w_next)
    sacc = _mp_call(g, src3, dst3, zeros_r)
    return _fin_call(sacc, g, dinv, b4.reshape(1, D), batch_t, mask_t,
                     w_att, b_att.reshape(1, 1), W_out, b_out.reshape(1, D))


# speedup vs baseline: 24.6904x; 1.1241x over previous
import functools

import jax
import jax.numpy as jnp
from jax import lax
from jax.experimental import pallas as pl
from jax.experimental.pallas import tpu as pltpu
from jax.experimental.pallas import tpu_sc as plsc

N = 10000
E = 320000
D = 128
B = 16
NP = 10240
NSC = 2
NSUB = 16
NW = NSC * NSUB
EPT = E // NW
K = 64
NCHUNK = 160
EPT2 = NCHUNK * K
NBUF = 4
WIN = 32
NWIN = NCHUNK // WIN
RPT = NP // NSUB
f32 = jnp.float32

_HI = lax.Precision.HIGHEST

_mesh = plsc.VectorSubcoreMesh(core_axis_name="c", subcore_axis_name="s",
                               num_cores=NSC, num_subcores=NSUB)


def _deg_body(dst_hbm, zeros_hbm, ones_hbm, out_hbm, accum, idx_v, ones_v):
    c = lax.axis_index("c")
    s = lax.axis_index("s")
    wid = c * NSUB + s
    pltpu.sync_copy(zeros_hbm, accum.at[pl.ds(s * RPT, RPT)])
    pltpu.sync_copy(dst_hbm.at[wid], idx_v)
    pltpu.sync_copy(ones_hbm, ones_v)
    plsc.subcore_barrier()

    def chunk(j, carry):
        pltpu.sync_copy(ones_v, accum.at[idx_v.at[j]], add=True)
        return carry

    lax.fori_loop(0, NCHUNK, chunk, 0)
    plsc.subcore_barrier()
    pltpu.sync_copy(accum.at[pl.ds(s * RPT, RPT)],
                    out_hbm.at[c, pl.ds(s * RPT, RPT)])


_deg_call = pl.kernel(
    _deg_body,
    out_type=jax.ShapeDtypeStruct((NSC, NP), f32),
    mesh=_mesh,
    scratch_types=[
        pltpu.VMEM_SHARED((NP,), f32),
        pltpu.VMEM((NCHUNK, K), jnp.int32),
        pltpu.VMEM((K,), f32),
    ],
)


def _mp_body(g_hbm, src_hbm, dst_hbm, zeros_hbm, out_hbm,
             accum, srcwin, dstwin, rowbuf, gsem, ssem):
    c = lax.axis_index("c")
    s = lax.axis_index("s")
    wid = c * NSUB + s
    zcp = pltpu.async_copy(zeros_hbm, accum.at[pl.ds(s * RPT, RPT)], ssem)
    pltpu.async_copy(src_hbm.at[wid, pl.ds(0, WIN)], srcwin.at[0], gsem)
    pltpu.async_copy(dst_hbm.at[wid, pl.ds(0, WIN)], dstwin.at[0], gsem)
    pltpu.make_async_copy(src_hbm.at[wid, pl.ds(0, WIN)], srcwin.at[0],
                          gsem).wait()
    pltpu.make_async_copy(dst_hbm.at[wid, pl.ds(0, WIN)], dstwin.at[0],
                          gsem).wait()
    zcp.wait()
    plsc.subcore_barrier()

    pltpu.async_copy(g_hbm.at[srcwin.at[0, 0]], rowbuf.at[0], gsem)
    pltpu.async_copy(g_hbm.at[srcwin.at[0, 1]], rowbuf.at[1], gsem)

    def window(w, carry):
        wslot = lax.rem(w, 2)
        nslot = lax.rem(w + 1, 2)

        @pl.when(w > 0)
        def _():
            for back in (2, 1):
                pltpu.make_async_copy(
                    rowbuf.at[lax.rem(w * WIN - back, NBUF)],
                    accum.at[dstwin.at[nslot, WIN - back]], ssem).wait()

        @pl.when(w + 1 < NWIN)
        def _():
            pltpu.sync_copy(src_hbm.at[wid, pl.ds((w + 1) * WIN, WIN)],
                            srcwin.at[nslot])
            pltpu.sync_copy(dst_hbm.at[wid, pl.ds((w + 1) * WIN, WIN)],
                            dstwin.at[nslot])

        for b in range(WIN):
            j = w * WIN + b
            slot = lax.rem(j, NBUF)
            pltpu.make_async_copy(g_hbm.at[srcwin.at[wslot, b]],
                                  rowbuf.at[slot], gsem).wait()
            pltpu.async_copy(rowbuf.at[slot], accum.at[dstwin.at[wslot, b]],
                             ssem, add=True)
            if b > 1:
                pltpu.make_async_copy(rowbuf.at[lax.rem(j - 2, NBUF)],
                                      accum.at[dstwin.at[wslot, b - 2]],
                                      ssem).wait()

            @pl.when(j + 2 < NCHUNK)
            def _():
                gw = lax.rem((j + 2) // WIN, 2)
                pltpu.async_copy(
                    g_hbm.at[srcwin.at[gw, (b + 2) % WIN]],
                    rowbuf.at[lax.rem(j + 2, NBUF)], gsem)

        return carry

    lax.fori_loop(0, NWIN, window, 0)
    for back in (2, 1):
        pltpu.make_async_copy(rowbuf.at[(NCHUNK - back) % NBUF],
                              accum.at[dstwin.at[(NWIN - 1) % 2, WIN - back]],
                              ssem).wait()
    plsc.subcore_barrier()
    pltpu.sync_copy(accum.at[pl.ds(s * RPT, RPT)],
                    out_hbm.at[c, pl.ds(s * RPT, RPT)])


_mp_call = pl.kernel(
    _mp_body,
    out_type=jax.ShapeDtypeStruct((NSC, NP, D), f32),
    mesh=_mesh,
    scratch_types=[
        pltpu.VMEM_SHARED((NP, D), f32),
        pltpu.VMEM((2, WIN, K), jnp.int32),
        pltpu.VMEM((2, WIN, K), jnp.int32),
        pltpu.VMEM((NBUF, K, D), f32),
        pltpu.SemaphoreType.DMA,
        pltpu.SemaphoreType.DMA,
    ],
)


def _pre_body(x_ref, deg_ref, w_ref, dinv_ref, g_ref):
    degp = deg_ref[...]
    rinv = lax.rsqrt(degp[0:1] + degp[1:2] + 1.0)
    dinv = lax.dot_general(rinv, jnp.ones((1, 1), f32),
                           (((0,), (0,)), ((), ())),
                           preferred_element_type=f32,
                           precision=_HI)
    dinv_ref[...] = dinv
    hw = jnp.dot(x_ref[...], w_ref[...],
                 preferred_element_type=f32, precision=_HI)
    g_ref[pl.ds(0, N), :] = hw * dinv[:N]
    g_ref[pl.ds(N, NP - N), :] = jnp.zeros((NP - N, D), f32)


_pre_call = pl.pallas_call(
    _pre_body,
    out_shape=[jax.ShapeDtypeStruct((NP, 1), f32),
               jax.ShapeDtypeStruct((NP, D), f32)],
)


def _mid_body(s_ref, g_ref, dinv_ref, b_ref, w_ref, out_ref):
    sacc = s_ref[...]
    dinv = dinv_ref[...]
    h = jnp.maximum(dinv * (sacc[0] + sacc[1] + g_ref[...]) + b_ref[...], 0.0)
    out_ref[...] = jnp.dot(h, w_ref[...],
                           preferred_element_type=f32, precision=_HI) * dinv


_mid_call = pl.pallas_call(
    _mid_body,
    out_shape=jax.ShapeDtypeStruct((NP, D), f32),
)


def _fin_body(s_ref, g_ref, dinv_ref, b_ref, batch_ref, mask_ref,
              watt_ref, batt_ref, wout_ref, bout_ref, out_ref):
    sacc = s_ref[...]
    h = jnp.maximum(dinv_ref[...] * (sacc[0] + sacc[1] + g_ref[...])
                    + b_ref[...], 0.0)
    bt = batch_ref[...]
    iot = lax.broadcasted_iota(jnp.int32, (B, NP), 0)
    mb = bt == iot
    mf = mb.astype(f32)
    counts = jnp.sum(mf, axis=1, keepdims=True)
    pooled = jnp.dot(mf, h, preferred_element_type=f32,
                     precision=_HI) / jnp.maximum(counts, 1.0)
    protein = pooled / jnp.sqrt(counts + 1e-6)
    scores = lax.dot_general(watt_ref[...], h, (((0,), (1,)), ((), ())),
                             preferred_element_type=f32,
                             precision=_HI) + batt_ref[...]
    mk = mask_ref[...] > 0
    neg = jnp.float32(-jnp.inf)
    masked = jnp.where(mb & mk, scores, neg)
    seg_max = jnp.max(masked, axis=1, keepdims=True)
    seg_max = jnp.where(seg_max > neg, seg_max, 0.0)
    smax_node = jnp.sum(mf * seg_max, axis=0, keepdims=True)
    ex = jnp.where(mk, jnp.exp(scores - smax_node), 0.0)
    seg_sum = jnp.sum(mf * ex, axis=1, keepdims=True)
    denom = jnp.where(seg_sum > 0, seg_sum, 1.0)
    den_node = jnp.sum(mf * denom, axis=0, keepdims=True)
    attn = ex / jnp.maximum(den_node, 1e-37)
    lys = jnp.dot(mf * attn, h, preferred_element_type=f32, precision=_HI)
    out_ref[...] = jnp.dot(protein + lys, wout_ref[...],
                           preferred_element_type=f32,
                           precision=_HI) + bout_ref[...]


_fin_call = pl.pallas_call(
    _fin_body,
    out_shape=jax.ShapeDtypeStruct((B, D), f32),
)


def kernel(x, edge_index, batch, lysine_mask, W1, b1, W2, b2, W3, b3,
           W4, b4, w_att, b_att, W_out, b_out):
    npad_rows = NP - N
    pad_iota = jnp.arange(EPT2 - EPT, dtype=jnp.int32)
    pad_src = jnp.broadcast_to(N + (pad_iota % npad_rows), (NW, EPT2 - EPT))
    pad_dst = jnp.broadcast_to(N + ((pad_iota + npad_rows // 2) % npad_rows),
                               (NW, EPT2 - EPT))
    src3 = jnp.concatenate(
        [edge_index[0].reshape(NW, EPT), pad_src], axis=1).reshape(
        NW, NCHUNK, K)
    dst3 = jnp.concatenate(
        [edge_index[1].reshape(NW, EPT), pad_dst], axis=1).reshape(
        NW, NCHUNK, K)
    batch_t = jnp.pad(batch, (0, NP - N), constant_values=B).reshape(1, NP)
    mask_t = jnp.pad(lysine_mask, (0, NP - N)).astype(f32).reshape(1, NP)
    zeros1 = jnp.zeros((RPT,), f32)
    ones1 = jnp.ones((K,), f32)
    zeros_r = jnp.zeros((RPT, D), f32)

    deg2 = _deg_call(dst3, zeros1, ones1)
    dinv, g = _pre_call(x, deg2, W1)
    for b_l, w_next in ((b1, W2), (b2, W3), (b3, W4)):
        sacc = _mp_call(g, src3, dst3, zeros_r)
        g = _mid_call(sacc, g, dinv, b_l.reshape(1, D), w_next)
    sacc = _mp_call(g, src3, dst3, zeros_r)
    return _fin_call(sacc, g, dinv, b4.reshape(1, D), batch_t, mask_t,
                     w_att, b_att.reshape(1, 1), W_out, b_out.reshape(1, D))


# speedup vs baseline: 25.3079x; 1.0250x over previous
import functools

import jax
import jax.numpy as jnp
from jax import lax
from jax.experimental import pallas as pl
from jax.experimental.pallas import tpu as pltpu
from jax.experimental.pallas import tpu_sc as plsc

N = 10000
E = 320000
D = 128
B = 16
NP = 10240
NSC = 2
NSUB = 16
NW = NSC * NSUB
EPT = E // NW
K = 64
NCHUNK = 160
EPT2 = NCHUNK * K
NBUF = 4
WIN = 32
NWIN = NCHUNK // WIN
RPT = NP // NSUB
f32 = jnp.float32

_HI = lax.Precision.HIGHEST

_mesh = plsc.VectorSubcoreMesh(core_axis_name="c", subcore_axis_name="s",
                               num_cores=NSC, num_subcores=NSUB)


def _deg_body(dst_hbm, zeros_hbm, ones_hbm, out_hbm, accum, idx_v, ones_v):
    c = lax.axis_index("c")
    s = lax.axis_index("s")
    wid = c * NSUB + s
    pltpu.sync_copy(zeros_hbm, accum.at[pl.ds(s * RPT, RPT)])
    pltpu.sync_copy(dst_hbm.at[wid], idx_v)
    pltpu.sync_copy(ones_hbm, ones_v)
    plsc.subcore_barrier()

    def chunk(j, carry):
        pltpu.sync_copy(ones_v, accum.at[idx_v.at[j]], add=True)
        return carry

    lax.fori_loop(0, NCHUNK, chunk, 0)
    plsc.subcore_barrier()
    pltpu.sync_copy(accum.at[pl.ds(s * RPT, RPT)],
                    out_hbm.at[c, pl.ds(s * RPT, RPT)])


_deg_call = pl.kernel(
    _deg_body,
    out_type=jax.ShapeDtypeStruct((NSC, NP), f32),
    mesh=_mesh,
    scratch_types=[
        pltpu.VMEM_SHARED((NP,), f32),
        pltpu.VMEM((NCHUNK, K), jnp.int32),
        pltpu.VMEM((K,), f32),
    ],
)


def _mp_body(g_hbm, src_hbm, dst_hbm, zeros_hbm, out_hbm,
             accum, srcwin, dstwin, rowbuf, gsem, ssem, isem):
    c = lax.axis_index("c")
    s = lax.axis_index("s")
    wid = c * NSUB + s
    zcp = pltpu.async_copy(zeros_hbm, accum.at[pl.ds(s * RPT, RPT)], ssem)
    pltpu.async_copy(src_hbm.at[wid, pl.ds(0, WIN)], srcwin.at[0], gsem)
    pltpu.async_copy(dst_hbm.at[wid, pl.ds(0, WIN)], dstwin.at[0], gsem)
    pltpu.make_async_copy(src_hbm.at[wid, pl.ds(0, WIN)], srcwin.at[0],
                          gsem).wait()
    pltpu.make_async_copy(dst_hbm.at[wid, pl.ds(0, WIN)], dstwin.at[0],
                          gsem).wait()
    zcp.wait()
    plsc.subcore_barrier()

    pltpu.async_copy(g_hbm.at[srcwin.at[0, 0]], rowbuf.at[0], gsem)
    pltpu.async_copy(g_hbm.at[srcwin.at[0, 1]], rowbuf.at[1], gsem)

    def window(w, carry):
        wslot = lax.rem(w, 2)
        nslot = lax.rem(w + 1, 2)

        @pl.when(w > 0)
        def _():
            for back in (2, 1):
                pltpu.make_async_copy(
                    rowbuf.at[lax.rem(w * WIN - back, NBUF)],
                    accum.at[dstwin.at[nslot, WIN - back]], ssem).wait()

        @pl.when(w + 1 < NWIN)
        def _():
            pltpu.async_copy(src_hbm.at[wid, pl.ds((w + 1) * WIN, WIN)],
                             srcwin.at[nslot], isem)
            pltpu.async_copy(dst_hbm.at[wid, pl.ds((w + 1) * WIN, WIN)],
                             dstwin.at[nslot], isem)

        for b in range(WIN):
            j = w * WIN + b
            slot = lax.rem(j, NBUF)
            if b == WIN - 2:
                @pl.when(w + 1 < NWIN)
                def _():
                    pltpu.make_async_copy(
                        src_hbm.at[wid, pl.ds((w + 1) * WIN, WIN)],
                        srcwin.at[nslot], isem).wait()
                    pltpu.make_async_copy(
                        dst_hbm.at[wid, pl.ds((w + 1) * WIN, WIN)],
                        dstwin.at[nslot], isem).wait()
            pltpu.make_async_copy(g_hbm.at[srcwin.at[wslot, b]],
                                  rowbuf.at[slot], gsem).wait()
            pltpu.async_copy(rowbuf.at[slot], accum.at[dstwin.at[wslot, b]],
                             ssem, add=True)
            if b > 1:
                pltpu.make_async_copy(rowbuf.at[lax.rem(j - 2, NBUF)],
                                      accum.at[dstwin.at[wslot, b - 2]],
                                      ssem).wait()

            @pl.when(j + 2 < NCHUNK)
            def _():
                gw = lax.rem((j + 2) // WIN, 2)
                pltpu.async_copy(
                    g_hbm.at[srcwin.at[gw, (b + 2) % WIN]],
                    rowbuf.at[lax.rem(j + 2, NBUF)], gsem)

        return carry

    lax.fori_loop(0, NWIN, window, 0)
    for back in (2, 1):
        pltpu.make_async_copy(rowbuf.at[(NCHUNK - back) % NBUF],
                              accum.at[dstwin.at[(NWIN - 1) % 2, WIN - back]],
                              ssem).wait()
    plsc.subcore_barrier()
    pltpu.sync_copy(accum.at[pl.ds(s * RPT, RPT)],
                    out_hbm.at[c, pl.ds(s * RPT, RPT)])


_mp_call = pl.kernel(
    _mp_body,
    out_type=jax.ShapeDtypeStruct((NSC, NP, D), f32),
    mesh=_mesh,
    scratch_types=[
        pltpu.VMEM_SHARED((NP, D), f32),
        pltpu.VMEM((2, WIN, K), jnp.int32),
        pltpu.VMEM((2, WIN, K), jnp.int32),
        pltpu.VMEM((NBUF, K, D), f32),
        pltpu.SemaphoreType.DMA,
        pltpu.SemaphoreType.DMA,
        pltpu.SemaphoreType.DMA,
    ],
)


def _pre_body(x_ref, deg_ref, w_ref, dinv_ref, g_ref):
    degp = deg_ref[...]
    rinv = lax.rsqrt(degp[0:1] + degp[1:2] + 1.0)
    dinv = lax.dot_general(rinv, jnp.ones((1, 1), f32),
                           (((0,), (0,)), ((), ())),
                           preferred_element_type=f32,
                           precision=_HI)
    dinv_ref[...] = dinv
    hw = jnp.dot(x_ref[...], w_ref[...],
                 preferred_element_type=f32, precision=_HI)
    g_ref[pl.ds(0, N), :] = hw * dinv[:N]
    g_ref[pl.ds(N, NP - N), :] = jnp.zeros((NP - N, D), f32)


_pre_call = pl.pallas_call(
    _pre_body,
    out_shape=[jax.ShapeDtypeStruct((NP, 1), f32),
               jax.ShapeDtypeStruct((NP, D), f32)],
)


def _mid_body(s_ref, g_ref, dinv_ref, b_ref, w_ref, out_ref):
    sacc = s_ref[...]
    dinv = dinv_ref[...]
    h = jnp.maximum(dinv * (sacc[0] + sacc[1] + g_ref[...]) + b_ref[...], 0.0)
    out_ref[...] = jnp.dot(h, w_ref[...],
                           preferred_element_type=f32, precision=_HI) * dinv


_mid_call = pl.pallas_call(
    _mid_body,
    out_shape=jax.ShapeDtypeStruct((NP, D), f32),
)


def _fin_body(s_ref, g_ref, dinv_ref, b_ref, batch_ref, mask_ref,
              watt_ref, batt_ref, wout_ref, bout_ref, out_ref):
    sacc = s_ref[...]
    h = jnp.maximum(dinv_ref[...] * (sacc[0] + sacc[1] + g_ref[...])
                    + b_ref[...], 0.0)
    bt = batch_ref[...]
    iot = lax.broadcasted_iota(jnp.int32, (B, NP), 0)
    mb = bt == iot
    mf = mb.astype(f32)
    counts = jnp.sum(mf, axis=1, keepdims=True)
    pooled = jnp.dot(mf, h, preferred_element_type=f32,
                     precision=_HI) / jnp.maximum(counts, 1.0)
    protein = pooled / jnp.sqrt(counts + 1e-6)
    scores = lax.dot_general(watt_ref[...], h, (((0,), (1,)), ((), ())),
                             preferred_element_type=f32,
                             precision=_HI) + batt_ref[...]
    mk = mask_ref[...] > 0
    neg = jnp.float32(-jnp.inf)
    masked = jnp.where(mb & mk, scores, neg)
    seg_max = jnp.max(masked, axis=1, keepdims=True)
    seg_max = jnp.where(seg_max > neg, seg_max, 0.0)
    smax_node = jnp.sum(mf * seg_max, axis=0, keepdims=True)
    ex = jnp.where(mk, jnp.exp(scores - smax_node), 0.0)
    seg_sum = jnp.sum(mf * ex, axis=1, keepdims=True)
    denom = jnp.where(seg_sum > 0, seg_sum, 1.0)
    den_node = jnp.sum(mf * denom, axis=0, keepdims=True)
    attn = ex / jnp.maximum(den_node, 1e-37)
    lys = jnp.dot(mf * attn, h, preferred_element_type=f32, precision=_HI)
    out_ref[...] = jnp.dot(protein + lys, wout_ref[...],
                           preferred_element_type=f32,
                           precision=_HI) + bout_ref[...]


_fin_call = pl.pallas_call(
    _fin_body,
    out_shape=jax.ShapeDtypeStruct((B, D), f32),
)


def kernel(x, edge_index, batch, lysine_mask, W1, b1, W2, b2, W3, b3,
           W4, b4, w_att, b_att, W_out, b_out):
    npad_rows = NP - N
    pad_iota = jnp.arange(EPT2 - EPT, dtype=jnp.int32)
    pad_src = jnp.broadcast_to(N + (pad_iota % npad_rows), (NW, EPT2 - EPT))
    pad_dst = jnp.broadcast_to(N + ((pad_iota + npad_rows // 2) % npad_rows),
                               (NW, EPT2 - EPT))
    src3 = jnp.concatenate(
        [edge_index[0].reshape(NW, EPT), pad_src], axis=1).reshape(
        NW, NCHUNK, K)
    dst3 = jnp.concatenate(
        [edge_index[1].reshape(NW, EPT), pad_dst], axis=1).reshape(
        NW, NCHUNK, K)
    batch_t = jnp.pad(batch, (0, NP - N), constant_values=B).reshape(1, NP)
    mask_t = jnp.pad(lysine_mask, (0, NP - N)).astype(f32).reshape(1, NP)
    zeros1 = jnp.zeros((RPT,), f32)
    ones1 = jnp.ones((K,), f32)
    zeros_r = jnp.zeros((RPT, D), f32)

    deg2 = _deg_call(dst3, zeros1, ones1)
    dinv, g = _pre_call(x, deg2, W1)
    for b_l, w_next in ((b1, W2), (b2, W3), (b3, W4)):
        sacc = _mp_call(g, src3, dst3, zeros_r)
        g = _mid_call(sacc, g, dinv, b_l.reshape(1, D), w_next)
    sacc = _mp_call(g, src3, dst3, zeros_r)
    return _fin_call(sacc, g, dinv, b4.reshape(1, D), batch_t, mask_t,
                     w_att, b_att.reshape(1, 1), W_out, b_out.reshape(1, D))


# speedup vs baseline: 25.8691x; 1.0222x over previous
import functools

import jax
import jax.numpy as jnp
from jax import lax
from jax.experimental import pallas as pl
from jax.experimental.pallas import tpu as pltpu
from jax.experimental.pallas import tpu_sc as plsc

N = 10000
E = 320000
D = 128
B = 16
NP = 10240
NSC = 2
NSUB = 16
NW = NSC * NSUB
EPT = E // NW
K = 64
NCHUNK = 160
EPT2 = NCHUNK * K
NBUF = 4
WIN = 32
NWIN = NCHUNK // WIN
RPT = NP // NSUB
f32 = jnp.float32

_HI = lax.Precision.HIGHEST

_mesh = plsc.VectorSubcoreMesh(core_axis_name="c", subcore_axis_name="s",
                               num_cores=NSC, num_subcores=NSUB)


def _deg_body(dst_hbm, zeros_hbm, ones_hbm, out_hbm, accum, idx_v, ones_v,
              ssem):
    c = lax.axis_index("c")
    s = lax.axis_index("s")
    wid = c * NSUB + s
    zcp = pltpu.async_copy(zeros_hbm, accum.at[pl.ds(s * RPT, RPT)], ssem)
    pltpu.sync_copy(dst_hbm.at[wid], idx_v)
    pltpu.sync_copy(ones_hbm, ones_v)
    zcp.wait()
    plsc.subcore_barrier()

    def chunk(j, carry):
        pltpu.async_copy(ones_v, accum.at[idx_v.at[j]], ssem, add=True)
        return carry

    lax.fori_loop(0, NCHUNK, chunk, 0)

    def drain(j, carry):
        pltpu.make_async_copy(ones_v, accum.at[idx_v.at[j]], ssem).wait()
        return carry

    lax.fori_loop(0, NCHUNK, drain, 0)
    plsc.subcore_barrier()
    pltpu.sync_copy(accum.at[pl.ds(s * RPT, RPT)],
                    out_hbm.at[c, pl.ds(s * RPT, RPT)])


_deg_call = pl.kernel(
    _deg_body,
    out_type=jax.ShapeDtypeStruct((NSC, NP), f32),
    mesh=_mesh,
    scratch_types=[
        pltpu.VMEM_SHARED((NP,), f32),
        pltpu.VMEM((NCHUNK, K), jnp.int32),
        pltpu.VMEM((K,), f32),
        pltpu.SemaphoreType.DMA,
    ],
)


def _mp_body(g_hbm, src_hbm, dst_hbm, zeros_hbm, out_hbm,
             accum, srcwin, dstwin, rowbuf, gsem, ssem, isem):
    c = lax.axis_index("c")
    s = lax.axis_index("s")
    wid = c * NSUB + s
    zcp = pltpu.async_copy(zeros_hbm, accum.at[pl.ds(s * RPT, RPT)], ssem)
    pltpu.async_copy(src_hbm.at[wid, pl.ds(0, WIN)], srcwin.at[0], gsem)
    pltpu.async_copy(dst_hbm.at[wid, pl.ds(0, WIN)], dstwin.at[0], gsem)
    pltpu.make_async_copy(src_hbm.at[wid, pl.ds(0, WIN)], srcwin.at[0],
                          gsem).wait()
    pltpu.make_async_copy(dst_hbm.at[wid, pl.ds(0, WIN)], dstwin.at[0],
                          gsem).wait()
    zcp.wait()
    plsc.subcore_barrier()

    pltpu.async_copy(g_hbm.at[srcwin.at[0, 0]], rowbuf.at[0], gsem)
    pltpu.async_copy(g_hbm.at[srcwin.at[0, 1]], rowbuf.at[1], gsem)

    def window(w, carry):
        wslot = lax.rem(w, 2)
        nslot = lax.rem(w + 1, 2)

        @pl.when(w > 0)
        def _():
            for back in (2, 1):
                pltpu.make_async_copy(
                    rowbuf.at[lax.rem(w * WIN - back, NBUF)],
                    accum.at[dstwin.at[nslot, WIN - back]], ssem).wait()

        @pl.when(w + 1 < NWIN)
        def _():
            pltpu.async_copy(src_hbm.at[wid, pl.ds((w + 1) * WIN, WIN)],
                             srcwin.at[nslot], isem)
            pltpu.async_copy(dst_hbm.at[wid, pl.ds((w + 1) * WIN, WIN)],
                             dstwin.at[nslot], isem)

        for b in range(WIN):
            j = w * WIN + b
            slot = lax.rem(j, NBUF)
            if b == WIN - 2:
                @pl.when(w + 1 < NWIN)
                def _():
                    pltpu.make_async_copy(
                        src_hbm.at[wid, pl.ds((w + 1) * WIN, WIN)],
                        srcwin.at[nslot], isem).wait()
                    pltpu.make_async_copy(
                        dst_hbm.at[wid, pl.ds((w + 1) * WIN, WIN)],
                        dstwin.at[nslot], isem).wait()
            pltpu.make_async_copy(g_hbm.at[srcwin.at[wslot, b]],
                                  rowbuf.at[slot], gsem).wait()
            pltpu.async_copy(rowbuf.at[slot], accum.at[dstwin.at[wslot, b]],
                             ssem, add=True)
            if b > 1:
                pltpu.make_async_copy(rowbuf.at[lax.rem(j - 2, NBUF)],
                                      accum.at[dstwin.at[wslot, b - 2]],
                                      ssem).wait()

            @pl.when(j + 2 < NCHUNK)
            def _():
                gw = lax.rem((j + 2) // WIN, 2)
                pltpu.async_copy(
                    g_hbm.at[srcwin.at[gw, (b + 2) % WIN]],
                    rowbuf.at[lax.rem(j + 2, NBUF)], gsem)

        return carry

    lax.fori_loop(0, NWIN, window, 0)
    for back in (2, 1):
        pltpu.make_async_copy(rowbuf.at[(NCHUNK - back) % NBUF],
                              accum.at[dstwin.at[(NWIN - 1) % 2, WIN - back]],
                              ssem).wait()
    plsc.subcore_barrier()
    pltpu.sync_copy(accum.at[pl.ds(s * RPT, RPT)],
                    out_hbm.at[c, pl.ds(s * RPT, RPT)])


_mp_call = pl.kernel(
    _mp_body,
    out_type=jax.ShapeDtypeStruct((NSC, NP, D), f32),
    mesh=_mesh,
    scratch_types=[
        pltpu.VMEM_SHARED((NP, D), f32),
        pltpu.VMEM((2, WIN, K), jnp.int32),
        pltpu.VMEM((2, WIN, K), jnp.int32),
        pltpu.VMEM((NBUF, K, D), f32),
        pltpu.SemaphoreType.DMA,
        pltpu.SemaphoreType.DMA,
        pltpu.SemaphoreType.DMA,
    ],
)


def _pre_body(x_ref, deg_ref, w_ref, dinv_ref, g_ref):
    degp = deg_ref[...]
    rinv = lax.rsqrt(degp[0:1] + degp[1:2] + 1.0)
    dinv = lax.dot_general(rinv, jnp.ones((1, 1), f32),
                           (((0,), (0,)), ((), ())),
                           preferred_element_type=f32,
                           precision=_HI)
    dinv_ref[...] = dinv
    hw = jnp.dot(x_ref[...], w_ref[...],
                 preferred_element_type=f32, precision=_HI)
    g_ref[pl.ds(0, N), :] = hw * dinv[:N]
    g_ref[pl.ds(N, NP - N), :] = jnp.zeros((NP - N, D), f32)


_pre_call = pl.pallas_call(
    _pre_body,
    out_shape=[jax.ShapeDtypeStruct((NP, 1), f32),
               jax.ShapeDtypeStruct((NP, D), f32)],
)


def _mid_body(s_ref, g_ref, dinv_ref, b_ref, w_ref, out_ref):
    sacc = s_ref[...]
    dinv = dinv_ref[...]
    h = jnp.maximum(dinv * (sacc[0] + sacc[1] + g_ref[...]) + b_ref[...], 0.0)
    out_ref[...] = jnp.dot(h, w_ref[...],
                           preferred_element_type=f32, precision=_HI) * dinv


_mid_call = pl.pallas_call(
    _mid_body,
    out_shape=jax.ShapeDtypeStruct((NP, D), f32),
)


def _fin_body(s_ref, g_ref, dinv_ref, b_ref, batch_ref, mask_ref,
              watt_ref, batt_ref, wout_ref, bout_ref, out_ref):
    sacc = s_ref[...]
    h = jnp.maximum(dinv_ref[...] * (sacc[0] + sacc[1] + g_ref[...])
                    + b_ref[...], 0.0)
    bt = batch_ref[...]
    iot = lax.broadcasted_iota(jnp.int32, (B, NP), 0)
    mb = bt == iot
    mf = mb.astype(f32)
    counts = jnp.sum(mf, axis=1, keepdims=True)
    pooled = jnp.dot(mf, h, preferred_element_type=f32,
                     precision=_HI) / jnp.maximum(counts, 1.0)
    protein = pooled / jnp.sqrt(counts + 1e-6)
    scores = lax.dot_general(watt_ref[...], h, (((0,), (1,)), ((), ())),
                             preferred_element_type=f32,
                             precision=_HI) + batt_ref[...]
    mk = mask_ref[...] > 0
    neg = jnp.float32(-jnp.inf)
    masked = jnp.where(mb & mk, scores, neg)
    seg_max = jnp.max(masked, axis=1, keepdims=True)
    seg_max = jnp.where(seg_max > neg, seg_max, 0.0)
    smax_node = jnp.sum(mf * seg_max, axis=0, keepdims=True)
    ex = jnp.where(mk, jnp.exp(scores - smax_node), 0.0)
    seg_sum = jnp.sum(mf * ex, axis=1, keepdims=True)
    denom = jnp.where(seg_sum > 0, seg_sum, 1.0)
    den_node = jnp.sum(mf * denom, axis=0, keepdims=True)
    attn = ex / jnp.maximum(den_node, 1e-37)
    lys = jnp.dot(mf * attn, h, preferred_element_type=f32, precision=_HI)
    out_ref[...] = jnp.dot(protein + lys, wout_ref[...],
                           preferred_element_type=f32,
                           precision=_HI) + bout_ref[...]


_fin_call = pl.pallas_call(
    _fin_body,
    out_shape=jax.ShapeDtypeStruct((B, D), f32),
)


def kernel(x, edge_index, batch, lysine_mask, W1, b1, W2, b2, W3, b3,
           W4, b4, w_att, b_att, W_out, b_out):
    npad_rows = NP - N
    pad_iota = jnp.arange(EPT2 - EPT, dtype=jnp.int32)
    pad_src = jnp.broadcast_to(N + (pad_iota % npad_rows), (NW, EPT2 - EPT))
    pad_dst = jnp.broadcast_to(N + ((pad_iota + npad_rows // 2) % npad_rows),
                               (NW, EPT2 - EPT))
    src3 = jnp.concatenate(
        [edge_index[0].reshape(NW, EPT), pad_src], axis=1).reshape(
        NW, NCHUNK, K)
    dst3 = jnp.concatenate(
        [edge_index[1].reshape(NW, EPT), pad_dst], axis=1).reshape(
        NW, NCHUNK, K)
    batch_t = jnp.pad(batch, (0, NP - N), constant_values=B).reshape(1, NP)
    mask_t = jnp.pad(lysine_mask, (0, NP - N)).astype(f32).reshape(1, NP)
    zeros1 = jnp.zeros((RPT,), f32)
    ones1 = jnp.ones((K,), f32)
    zeros_r = jnp.zeros((RPT, D), f32)

    deg2 = _deg_call(dst3, zeros1, ones1)
    dinv, g = _pre_call(x, deg2, W1)
    for b_l, w_next in ((b1, W2), (b2, W3), (b3, W4)):
        sacc = _mp_call(g, src3, dst3, zeros_r)
        g = _mid_call(sacc, g, dinv, b_l.reshape(1, D), w_next)
    sacc = _mp_call(g, src3, dst3, zeros_r)
    return _fin_call(sacc, g, dinv, b4.reshape(1, D), batch_t, mask_t,
                     w_att, b_att.reshape(1, 1), W_out, b_out.reshape(1, D))
